# Initial kernel scaffold; baseline (speedup 1.0000x reference)
import functools

import jax
import jax.numpy as jnp
from jax import lax
from jax.experimental import pallas as pl
from jax.experimental.pallas import tpu as pltpu
from jax.experimental.pallas import tpu_sc as plsc

N = 10000
E = 320000
F = 128
NC = 2
NS = 16
CHUNK = 80
EPT = E // NS
NCHUNK = EPT // CHUNK
ROWS_PT = N // NS
ZROWS = 125

_mesh = plsc.VectorSubcoreMesh(core_axis_name="c", subcore_axis_name="s")


@functools.partial(
    pl.kernel,
    out_type=jax.ShapeDtypeStruct((NC, N, F), jnp.float32),
    mesh=_mesh,
    scratch_types=[
        pltpu.VMEM((NCHUNK, CHUNK), jnp.int32),
        pltpu.VMEM((NCHUNK, CHUNK), jnp.int32),
        pltpu.VMEM((CHUNK, F), jnp.float32),
        pltpu.VMEM((CHUNK, F), jnp.float32),
        pltpu.VMEM((ZROWS, F), jnp.float32),
        pltpu.VMEM_SHARED((N, F), jnp.float32),
        pltpu.SemaphoreType.DMA,
        pltpu.SemaphoreType.DMA,
    ],
)
def _sc_aggregate(x_hbm, eidx_hbm, out_hbm, src_v, dst_v, rows0, rows1,
                  zbuf, acc_sh, sem0, sem1):
    c = lax.axis_index("c")
    s = lax.axis_index("s")

    def zrow(i, carry):
        def zcol(k, carry2):
            zbuf[i, pl.ds(k * 16, 16)] = jnp.zeros((16,), jnp.float32)
            return carry2
        return lax.fori_loop(0, F // 16, zcol, carry)
    lax.fori_loop(0, ZROWS, zrow, 0)
    for j in range(ROWS_PT // ZROWS):
        pltpu.sync_copy(zbuf, acc_sh.at[pl.ds(s * ROWS_PT + j * ZROWS, ZROWS)])
    plsc.subcore_barrier()

    pltpu.sync_copy(eidx_hbm.at[c, 0, s], src_v)
    pltpu.sync_copy(eidx_hbm.at[c, 1, s], dst_v)

    pltpu.async_copy(x_hbm.at[src_v.at[0]], rows0, sem0)

    def body(k, carry):
        j0 = 2 * k
        pltpu.async_copy(x_hbm.at[src_v.at[j0 + 1]], rows1, sem1)
        pltpu.make_async_copy(x_hbm.at[src_v.at[j0]], rows0, sem0).wait()
        pltpu.sync_copy(rows0, acc_sh.at[dst_v.at[j0]], add=True)

        @pl.when(k < NCHUNK // 2 - 1)
        def _():
            pltpu.async_copy(x_hbm.at[src_v.at[j0 + 2]], rows0, sem0)
        pltpu.make_async_copy(x_hbm.at[src_v.at[j0 + 1]], rows1, sem1).wait()
        pltpu.sync_copy(rows1, acc_sh.at[dst_v.at[j0 + 1]], add=True)
        return carry

    lax.fori_loop(0, NCHUNK // 2, body, 0)
    plsc.subcore_barrier()

    pltpu.sync_copy(acc_sh.at[pl.ds(s * ROWS_PT, ROWS_PT)],
                    out_hbm.at[c, pl.ds(s * ROWS_PT, ROWS_PT)])


def _mm_body(x_ref, a0_ref, a1_ref, wt_ref, o_ref):
    o_ref[...] = (
        jnp.dot(x_ref[...], wt_ref[0:F, :], preferred_element_type=jnp.float32)
        + jnp.dot(a0_ref[...], wt_ref[F:2 * F, :], preferred_element_type=jnp.float32)
        + jnp.dot(a1_ref[...], wt_ref[2 * F:3 * F, :], preferred_element_type=jnp.float32)
    )


_MM_BLK = 400

_mm_call = pl.pallas_call(
    _mm_body,
    out_shape=jax.ShapeDtypeStruct((N, F), jnp.float32),
    grid=(N // _MM_BLK,),
    in_specs=[
        pl.BlockSpec((_MM_BLK, F), lambda i: (i, 0)),
        pl.BlockSpec((_MM_BLK, F), lambda i: (i, 0)),
        pl.BlockSpec((_MM_BLK, F), lambda i: (i, 0)),
        pl.BlockSpec((3 * F, F), lambda i: (0, 0)),
    ],
    out_specs=pl.BlockSpec((_MM_BLK, F), lambda i: (i, 0)),
)


@jax.jit
def kernel(X, W, edge_index_0, edge_index_1):
    eidx = jnp.stack([edge_index_0.astype(jnp.int32),
                      edge_index_1.astype(jnp.int32)])
    eidx = eidx.reshape(NC, 2, NS, NCHUNK, CHUNK)
    agg = _sc_aggregate(X, eidx)
    return _mm_call(X, agg[0], agg[1], W.T)


# speedup vs baseline: 9.3282x; 9.3282x over previous
import functools

import jax
import jax.numpy as jnp
from jax import lax
from jax.experimental import pallas as pl
from jax.experimental.pallas import tpu as pltpu
from jax.experimental.pallas import tpu_sc as plsc

N = 10000
E = 320000
F = 128
NC = 2
NS = 16
CHUNK = 80
SEGC = 10
EPT = E // NS
NCHUNK = EPT // CHUNK
NSEG = NCHUNK // SEGC
NZB = N // CHUNK

_mesh = plsc.VectorSubcoreMesh(core_axis_name="c", subcore_axis_name="s")


@functools.partial(
    pl.kernel,
    out_type=jax.ShapeDtypeStruct((NC, N, F), jnp.float32),
    mesh=_mesh,
    scratch_types=[
        pltpu.VMEM((SEGC, CHUNK), jnp.int32),
        pltpu.VMEM((SEGC, CHUNK), jnp.int32),
        pltpu.VMEM((CHUNK, F), jnp.float32),
        pltpu.VMEM((CHUNK, F), jnp.float32),
        pltpu.VMEM_SHARED((N, F), jnp.float32),
        pltpu.SemaphoreType.DMA,
        pltpu.SemaphoreType.DMA,
    ],
)
def _sc_aggregate(x_hbm, eidx_hbm, out_hbm, src_v, dst_v, rows0, rows1,
                  acc_sh, sem0, sem1):
    c = lax.axis_index("c")
    s = lax.axis_index("s")

    def zrow(i, carry):
        def zcol(k, carry2):
            rows0[i, pl.ds(k * 16, 16)] = jnp.zeros((16,), jnp.float32)
            return carry2
        return lax.fori_loop(0, F // 16, zcol, carry)
    lax.fori_loop(0, CHUNK, zrow, 0)
    for t in range((NZB + NS - 1) // NS):
        blk = s + NS * t

        @pl.when(blk < NZB)
        def _():
            pltpu.sync_copy(rows0, acc_sh.at[pl.ds(blk * CHUNK, CHUNK)])
    plsc.subcore_barrier()

    def seg_body(seg, carry):
        pltpu.sync_copy(eidx_hbm.at[c, 0, s, seg], src_v)
        pltpu.sync_copy(eidx_hbm.at[c, 1, s, seg], dst_v)
        pltpu.async_copy(x_hbm.at[src_v.at[0]], rows0, sem0)
        for p in range(SEGC // 2):
            j0 = 2 * p
            pltpu.async_copy(x_hbm.at[src_v.at[j0 + 1]], rows1, sem1)
            pltpu.make_async_copy(x_hbm.at[src_v.at[j0]], rows0, sem0).wait()
            pltpu.sync_copy(rows0, acc_sh.at[dst_v.at[j0]], add=True)
            if j0 + 2 < SEGC:
                pltpu.async_copy(x_hbm.at[src_v.at[j0 + 2]], rows0, sem0)
            pltpu.make_async_copy(x_hbm.at[src_v.at[j0 + 1]], rows1, sem1).wait()
            pltpu.sync_copy(rows1, acc_sh.at[dst_v.at[j0 + 1]], add=True)
        return carry

    lax.fori_loop(0, NSEG, seg_body, 0)
    plsc.subcore_barrier()

    for t in range((NZB + NS - 1) // NS):
        blk = s + NS * t

        @pl.when(blk < NZB)
        def _():
            pltpu.sync_copy(acc_sh.at[pl.ds(blk * CHUNK, CHUNK)],
                            out_hbm.at[c, pl.ds(blk * CHUNK, CHUNK)])


def _mm_body(x_ref, a0_ref, a1_ref, wt_ref, o_ref):
    o_ref[...] = (
        jnp.dot(x_ref[...], wt_ref[0:F, :], preferred_element_type=jnp.float32)
        + jnp.dot(a0_ref[...], wt_ref[F:2 * F, :], preferred_element_type=jnp.float32)
        + jnp.dot(a1_ref[...], wt_ref[2 * F:3 * F, :], preferred_element_type=jnp.float32)
    )


_MM_BLK = 400

_mm_call = pl.pallas_call(
    _mm_body,
    out_shape=jax.ShapeDtypeStruct((N, F), jnp.float32),
    grid=(N // _MM_BLK,),
    in_specs=[
        pl.BlockSpec((_MM_BLK, F), lambda i: (i, 0)),
        pl.BlockSpec((_MM_BLK, F), lambda i: (i, 0)),
        pl.BlockSpec((_MM_BLK, F), lambda i: (i, 0)),
        pl.BlockSpec((3 * F, F), lambda i: (0, 0)),
    ],
    out_specs=pl.BlockSpec((_MM_BLK, F), lambda i: (i, 0)),
)


@jax.jit
def kernel(X, W, edge_index_0, edge_index_1):
    eidx = jnp.stack([edge_index_0.astype(jnp.int32),
                      edge_index_1.astype(jnp.int32)])
    eidx = eidx.reshape(NC, 2, NS, NSEG, SEGC, CHUNK)
    agg = _sc_aggregate(X, eidx)
    return _mm_call(X, agg[0], agg[1], W.T)
